# baseline (speedup 1.0000x reference)
import functools

import jax
import jax.numpy as jnp
from jax import lax
from jax.experimental import pallas as pl
from jax.experimental.pallas import tpu as pltpu
from jax.experimental.pallas import tpu_sc as plsc

N_NODES = 10000
N_EDGES = 320000
DIM = 128
N_TYPES = 100
E_TYPES = 400
N_CENTERS = 30
CUT_LOW, CUT_HIGH = 0.0, 10.0
N_CONV = 3

F32 = jnp.float32

_MESH = plsc.VectorSubcoreMesh(core_axis_name="core", subcore_axis_name="subcore")
NC = 2
NS = 16
NW = NC * NS


def _sc_gather(table, idx_1d, window):
    b_total = idx_1d.shape[0]
    d = table.shape[1]
    idx2 = idx_1d.reshape(1, b_total)

    @functools.partial(
        pl.kernel,
        out_type=jax.ShapeDtypeStruct((b_total, d), table.dtype),
        mesh=_MESH,
    )
    def k(tab_hbm, i_hbm, o_hbm):
        def body(i_vmem, o_vmem):
            pltpu.sync_copy(tab_hbm.at[i_vmem.at[0]], o_vmem)

        pltpu.emit_pipeline(
            body,
            grid=(b_total // window,),
            in_specs=[pl.BlockSpec((1, window), lambda i: (0, i))],
            out_specs=[pl.BlockSpec((window, d), lambda i: (i, 0))],
            core_axis_name=("core", "subcore"),
            dimension_semantics=(pltpu.PARALLEL,),
        )(i_hbm, o_hbm)

    return k(table, idx2)


def _sc_segment_partials(m, dst, zeros_nd):
    ch = 80
    per_w = N_EDGES // NW
    nch = per_w // ch
    rps_main, rps_last = 640, N_NODES - 15 * 640

    @functools.partial(
        pl.kernel,
        out_type=jax.ShapeDtypeStruct((NC, N_NODES, DIM), F32),
        mesh=_MESH,
        scratch_types=[
            pltpu.VMEM((ch, DIM), F32),
            pltpu.VMEM((ch,), jnp.int32),
            pltpu.VMEM_SHARED((N_NODES, DIM), F32),
        ],
    )
    def k(m_hbm, dst_hbm, z_hbm, out_hbm, rows_v, idx_v, acc_sh):
        cid = lax.axis_index("core")
        sid = lax.axis_index("subcore")
        wid = cid * NS + sid
        @pl.when(sid < 15)
        def _():
            pltpu.sync_copy(
                z_hbm.at[pl.ds(sid * rps_main, rps_main)],
                acc_sh.at[pl.ds(sid * rps_main, rps_main)],
            )

        @pl.when(sid == 15)
        def _():
            pltpu.sync_copy(
                z_hbm.at[pl.ds(15 * rps_main, rps_last)],
                acc_sh.at[pl.ds(15 * rps_main, rps_last)],
            )

        plsc.subcore_barrier()

        base = wid * per_w

        @pl.loop(0, nch)
        def _(j):
            off = base + j * ch
            pltpu.sync_copy(m_hbm.at[pl.ds(off, ch)], rows_v)
            pltpu.sync_copy(dst_hbm.at[pl.ds(off, ch)], idx_v)
            pltpu.sync_copy(rows_v, acc_sh.at[idx_v], add=True)

        plsc.subcore_barrier()

        @pl.when(sid < 15)
        def _():
            pltpu.sync_copy(
                acc_sh.at[pl.ds(sid * rps_main, rps_main)],
                out_hbm.at[cid].at[pl.ds(sid * rps_main, rps_main)],
            )

        @pl.when(sid == 15)
        def _():
            pltpu.sync_copy(
                acc_sh.at[pl.ds(15 * rps_main, rps_last)],
                out_hbm.at[cid].at[pl.ds(15 * rps_main, rps_last)],
            )

    return k(m, dst, zeros_nd)


def _tprep(edge_emb, We1, be1):
    def body(ee_ref, we1_ref, be1_ref, out_ref):
        ee = ee_ref[...]
        for i in range(N_CONV):
            t = jnp.dot(ee, we1_ref[i, :DIM, :], preferred_element_type=F32)
            out_ref[:, i * DIM:(i + 1) * DIM] = t + be1_ref[i, :][None, :]

    return pl.pallas_call(
        body,
        out_shape=jax.ShapeDtypeStruct((E_TYPES, N_CONV * DIM), F32),
    )(edge_emb, We1, be1)


def _hn_first(node_type_2d, node_emb, Wn1, bn1, Wn2, bn2):
    def body(nt_ref, emb_ref, w1_ref, b1_ref, w2_ref, b2_ref, h_ref, hn_ref):
        nt = nt_ref[...]
        ids = lax.broadcasted_iota(jnp.int32, (1, N_TYPES), 1)
        oh = (nt == ids).astype(F32)
        h0 = jnp.dot(oh, emb_ref[...], preferred_element_type=F32,
                     precision=jax.lax.Precision.HIGHEST)
        t = jnp.maximum(jnp.dot(h0, w1_ref[...], preferred_element_type=F32)
                        + b1_ref[...], 0.0)
        hn = jnp.dot(t, w2_ref[...], preferred_element_type=F32) + b2_ref[...]
        h_ref[...] = h0
        hn_ref[...] = hn

    return pl.pallas_call(
        body,
        out_shape=(
            jax.ShapeDtypeStruct((N_NODES, DIM), F32),
            jax.ShapeDtypeStruct((N_NODES, DIM), F32),
        ),
    )(node_type_2d, node_emb, Wn1, bn1.reshape(1, DIM), Wn2, bn2.reshape(1, DIM))


def _hn_next(h_prev, partials, Wn1, bn1, Wn2, bn2):
    def body(h_ref, p_ref, w1_ref, b1_ref, w2_ref, b2_ref, h_out, hn_ref):
        h = h_ref[...] + p_ref[0] + p_ref[1]
        t = jnp.maximum(jnp.dot(h, w1_ref[...], preferred_element_type=F32)
                        + b1_ref[...], 0.0)
        hn = jnp.dot(t, w2_ref[...], preferred_element_type=F32) + b2_ref[...]
        h_out[...] = h
        hn_ref[...] = hn

    return pl.pallas_call(
        body,
        out_shape=(
            jax.ShapeDtypeStruct((N_NODES, DIM), F32),
            jax.ShapeDtypeStruct((N_NODES, DIM), F32),
        ),
    )(h_prev, partials, Wn1, bn1.reshape(1, DIM), Wn2, bn2.reshape(1, DIM))


_BE = 512


def _messages(layer, te_all, dist3, hs, w1b, We2, be2, Wc, bc):
    gap = (CUT_HIGH - CUT_LOW) / N_CENTERS
    inv_gap2 = 1.0 / (gap * gap)
    BF = jnp.bfloat16

    def body(te_ref, d_ref, hs_ref, w1b_ref, w2_ref, b2_ref, wc_ref, bc_ref,
             m_ref):
        step = (CUT_HIGH - CUT_LOW) / (N_CENTERS - 1)
        centers_t = (lax.broadcasted_iota(jnp.int32, (N_CENTERS, 1), 0)
                     .astype(F32) * step + CUT_LOW)
        d = d_ref[0]
        delta = d - centers_t
        rbf_t = jnp.exp(-(delta * delta) * inv_gap2)
        r = lax.dot_general(rbf_t, w1b_ref[...],
                            (((0,), (0,)), ((), ())),
                            preferred_element_type=F32)
        pre = te_ref[...] + r
        he = jnp.dot(jnp.maximum(pre, 0.0), w2_ref[...],
                     preferred_element_type=F32) + b2_ref[...]
        prod = hs_ref[...] * he
        m_ref[...] = jnp.tanh(
            jnp.dot(prod, wc_ref[...],
                    preferred_element_type=F32) + bc_ref[...])

    nb = N_EDGES // _BE
    full = lambda *s: pl.BlockSpec(s, lambda b: tuple(0 for _ in s))
    return pl.pallas_call(
        body,
        grid=(nb,),
        in_specs=[
            pl.BlockSpec((_BE, DIM), lambda b: (b, layer)),
            pl.BlockSpec((1, 1, _BE), lambda b: (b, 0, 0)),
            pl.BlockSpec((_BE, DIM), lambda b: (b, 0)),
            full(N_CENTERS, DIM),
            full(DIM, DIM),
            full(1, DIM),
            full(DIM, DIM),
            full(1, DIM),
        ],
        out_specs=pl.BlockSpec((_BE, DIM), lambda b: (b, 0)),
        out_shape=jax.ShapeDtypeStruct((N_EDGES, DIM), F32),
    )(te_all, dist3, hs, w1b, We2, be2.reshape(1, DIM), Wc, bc.reshape(1, DIM))


def _readout(h_prev, partials, Wr1, br1, Wr2, br2):
    def body(h_ref, p_ref, w1_ref, b1_ref, w2_ref, b2_ref, out_ref):
        h = h_ref[...] + p_ref[0] + p_ref[1]
        t = jnp.maximum(jnp.dot(h, w1_ref[...], preferred_element_type=F32)
                        + b1_ref[...], 0.0)
        r = jnp.dot(t, w2_ref[...], preferred_element_type=F32) + b2_ref[...]
        out_ref[...] = jnp.sum(r, axis=0, keepdims=True)

    return pl.pallas_call(
        body,
        out_shape=jax.ShapeDtypeStruct((1, 1), F32),
    )(h_prev, partials, Wr1, br1.reshape(1, DIM), Wr2, br2.reshape(1, 1))


def kernel(node_type, edge_type, edge_index, dist, node_emb, edge_emb,
           Wn1, bn1, Wn2, bn2, We1, be1, We2, be2, Wc, bc,
           Wr1, br1, Wr2, br2):
    src = edge_index[0].astype(jnp.int32)
    dst = edge_index[1].astype(jnp.int32)
    etype = edge_type.astype(jnp.int32)
    nt2 = node_type.astype(jnp.int32).reshape(N_NODES, 1)
    dist3 = dist.reshape(N_EDGES // _BE, 1, _BE)
    zeros_nd = jnp.zeros((N_NODES, DIM), F32)

    t_all = _tprep(edge_emb, We1, be1)
    te_all = _sc_gather(t_all, etype, window=128)

    h, hn = _hn_first(nt2, node_emb, Wn1[0], bn1[0], Wn2[0], bn2[0])
    for i in range(N_CONV):
        hs = _sc_gather(hn, src, window=128)
        w1b = We1[i, DIM:DIM + N_CENTERS, :]
        m = _messages(i, te_all, dist3, hs, w1b, We2[i], be2[i], Wc[i], bc[i])
        partials = _sc_segment_partials(m, dst, zeros_nd)
        if i + 1 < N_CONV:
            h, hn = _hn_next(h, partials, Wn1[i + 1], bn1[i + 1],
                             Wn2[i + 1], bn2[i + 1])
    out = _readout(h, partials, Wr1, br1, Wr2, br2)
    return out.reshape(-1)


# speedup vs baseline: 1.1643x; 1.1643x over previous
import functools

import jax
import jax.numpy as jnp
from jax import lax
from jax.experimental import pallas as pl
from jax.experimental.pallas import tpu as pltpu
from jax.experimental.pallas import tpu_sc as plsc

N_NODES = 10000
N_EDGES = 320000
DIM = 128
N_TYPES = 100
E_TYPES = 400
N_CENTERS = 30
CUT_LOW, CUT_HIGH = 0.0, 10.0
N_CONV = 3

F32 = jnp.float32

_MESH = plsc.VectorSubcoreMesh(core_axis_name="core", subcore_axis_name="subcore")
NC = 2
NS = 16
NW = NC * NS


def _sc_gather(table, idx_1d, window):
    b_total = idx_1d.shape[0]
    d = table.shape[1]
    idx2 = idx_1d.reshape(1, b_total)

    @functools.partial(
        pl.kernel,
        out_type=jax.ShapeDtypeStruct((b_total, d), table.dtype),
        mesh=_MESH,
    )
    def k(tab_hbm, i_hbm, o_hbm):
        def body(i_vmem, o_vmem):
            pltpu.sync_copy(tab_hbm.at[i_vmem.at[0]], o_vmem)

        pltpu.emit_pipeline(
            body,
            grid=(b_total // window,),
            in_specs=[pl.BlockSpec((1, window), lambda i: (0, i))],
            out_specs=[pl.BlockSpec((window, d), lambda i: (i, 0))],
            core_axis_name=("core", "subcore"),
            dimension_semantics=(pltpu.PARALLEL,),
        )(i_hbm, o_hbm)

    return k(table, idx2)


def _sc_segment_partials(m, dst, zeros_nd):
    ch = 80
    per_w = N_EDGES // NW
    nch = per_w // ch
    rps_main, rps_last = 640, N_NODES - 15 * 640

    @functools.partial(
        pl.kernel,
        out_type=jax.ShapeDtypeStruct((NC, N_NODES, DIM), F32),
        mesh=_MESH,
        scratch_types=[
            pltpu.VMEM((ch, DIM), F32),
            pltpu.VMEM((ch,), jnp.int32),
            pltpu.VMEM_SHARED((N_NODES, DIM), F32),
        ],
    )
    def k(m_hbm, dst_hbm, z_hbm, out_hbm, rows_v, idx_v, acc_sh):
        cid = lax.axis_index("core")
        sid = lax.axis_index("subcore")
        wid = cid * NS + sid
        @pl.when(sid < 15)
        def _():
            pltpu.sync_copy(
                z_hbm.at[pl.ds(sid * rps_main, rps_main)],
                acc_sh.at[pl.ds(sid * rps_main, rps_main)],
            )

        @pl.when(sid == 15)
        def _():
            pltpu.sync_copy(
                z_hbm.at[pl.ds(15 * rps_main, rps_last)],
                acc_sh.at[pl.ds(15 * rps_main, rps_last)],
            )

        plsc.subcore_barrier()

        base = wid * per_w

        @pl.loop(0, nch)
        def _(j):
            off = base + j * ch
            pltpu.sync_copy(m_hbm.at[pl.ds(off, ch)], rows_v)
            pltpu.sync_copy(dst_hbm.at[pl.ds(off, ch)], idx_v)
            pltpu.sync_copy(rows_v, acc_sh.at[idx_v], add=True)

        plsc.subcore_barrier()

        @pl.when(sid < 15)
        def _():
            pltpu.sync_copy(
                acc_sh.at[pl.ds(sid * rps_main, rps_main)],
                out_hbm.at[cid].at[pl.ds(sid * rps_main, rps_main)],
            )

        @pl.when(sid == 15)
        def _():
            pltpu.sync_copy(
                acc_sh.at[pl.ds(15 * rps_main, rps_last)],
                out_hbm.at[cid].at[pl.ds(15 * rps_main, rps_last)],
            )

    return k(m, dst, zeros_nd)


def _tprep(edge_emb, We1, be1):
    def body(ee_ref, we1_ref, be1_ref, out_ref):
        ee = ee_ref[...]
        for i in range(N_CONV):
            t = jnp.dot(ee, we1_ref[i, :DIM, :], preferred_element_type=F32)
            out_ref[:, i * DIM:(i + 1) * DIM] = t + be1_ref[i, :][None, :]

    return pl.pallas_call(
        body,
        out_shape=jax.ShapeDtypeStruct((E_TYPES, N_CONV * DIM), F32),
    )(edge_emb, We1, be1)


def _hn_first(node_type_2d, node_emb, Wn1, bn1, Wn2, bn2):
    def body(nt_ref, emb_ref, w1_ref, b1_ref, w2_ref, b2_ref, h_ref, hn_ref):
        nt = nt_ref[...]
        ids = lax.broadcasted_iota(jnp.int32, (1, N_TYPES), 1)
        oh = (nt == ids).astype(F32)
        h0 = jnp.dot(oh, emb_ref[...], preferred_element_type=F32,
                     precision=jax.lax.Precision.HIGHEST)
        t = jnp.maximum(jnp.dot(h0, w1_ref[...], preferred_element_type=F32)
                        + b1_ref[...], 0.0)
        hn = jnp.dot(t, w2_ref[...], preferred_element_type=F32) + b2_ref[...]
        h_ref[...] = h0
        hn_ref[...] = hn

    return pl.pallas_call(
        body,
        out_shape=(
            jax.ShapeDtypeStruct((N_NODES, DIM), F32),
            jax.ShapeDtypeStruct((N_NODES, DIM), F32),
        ),
    )(node_type_2d, node_emb, Wn1, bn1.reshape(1, DIM), Wn2, bn2.reshape(1, DIM))


def _hn_next(h_prev, partials, Wn1, bn1, Wn2, bn2):
    def body(h_ref, p_ref, w1_ref, b1_ref, w2_ref, b2_ref, h_out, hn_ref):
        h = h_ref[...] + p_ref[0] + p_ref[1]
        t = jnp.maximum(jnp.dot(h, w1_ref[...], preferred_element_type=F32)
                        + b1_ref[...], 0.0)
        hn = jnp.dot(t, w2_ref[...], preferred_element_type=F32) + b2_ref[...]
        h_out[...] = h
        hn_ref[...] = hn

    return pl.pallas_call(
        body,
        out_shape=(
            jax.ShapeDtypeStruct((N_NODES, DIM), F32),
            jax.ShapeDtypeStruct((N_NODES, DIM), F32),
        ),
    )(h_prev, partials, Wn1, bn1.reshape(1, DIM), Wn2, bn2.reshape(1, DIM))


_BE = 1280


def _messages(layer, te_all, dist3, hs, w1b, We2, be2, Wc, bc):
    gap = (CUT_HIGH - CUT_LOW) / N_CENTERS
    inv_gap2 = 1.0 / (gap * gap)
    BF = jnp.bfloat16

    def body(te_ref, d_ref, hs_ref, w1b_ref, w2_ref, b2_ref, wc_ref, bc_ref,
             m_ref):
        step = (CUT_HIGH - CUT_LOW) / (N_CENTERS - 1)
        centers_t = (lax.broadcasted_iota(jnp.int32, (N_CENTERS, 1), 0)
                     .astype(F32) * step + CUT_LOW)
        d = d_ref[0]
        delta = d - centers_t
        rbf_t = jnp.exp(-(delta * delta) * inv_gap2)
        r = lax.dot_general(rbf_t, w1b_ref[...],
                            (((0,), (0,)), ((), ())),
                            preferred_element_type=F32)
        pre = te_ref[...] + r
        he = jnp.dot(jnp.maximum(pre, 0.0), w2_ref[...],
                     preferred_element_type=F32) + b2_ref[...]
        prod = hs_ref[...] * he
        m_ref[...] = jnp.tanh(
            jnp.dot(prod, wc_ref[...],
                    preferred_element_type=F32) + bc_ref[...])

    nb = N_EDGES // _BE
    full = lambda *s: pl.BlockSpec(s, lambda b: tuple(0 for _ in s))
    return pl.pallas_call(
        body,
        grid=(nb,),
        in_specs=[
            pl.BlockSpec((_BE, DIM), lambda b: (b, 0)),
            pl.BlockSpec((1, 1, _BE), lambda b: (b, 0, 0)),
            pl.BlockSpec((_BE, DIM), lambda b: (b, 0)),
            full(N_CENTERS, DIM),
            full(DIM, DIM),
            full(1, DIM),
            full(DIM, DIM),
            full(1, DIM),
        ],
        out_specs=pl.BlockSpec((_BE, DIM), lambda b: (b, 0)),
        out_shape=jax.ShapeDtypeStruct((N_EDGES, DIM), F32),
    )(te_all, dist3, hs, w1b, We2, be2.reshape(1, DIM), Wc, bc.reshape(1, DIM))


def _readout(h_prev, partials, Wr1, br1, Wr2, br2):
    def body(h_ref, p_ref, w1_ref, b1_ref, w2_ref, b2_ref, out_ref):
        h = h_ref[...] + p_ref[0] + p_ref[1]
        t = jnp.maximum(jnp.dot(h, w1_ref[...], preferred_element_type=F32)
                        + b1_ref[...], 0.0)
        r = jnp.dot(t, w2_ref[...], preferred_element_type=F32) + b2_ref[...]
        out_ref[...] = jnp.sum(r, axis=0, keepdims=True)

    return pl.pallas_call(
        body,
        out_shape=jax.ShapeDtypeStruct((1, 1), F32),
    )(h_prev, partials, Wr1, br1.reshape(1, DIM), Wr2, br2.reshape(1, 1))


def kernel(node_type, edge_type, edge_index, dist, node_emb, edge_emb,
           Wn1, bn1, Wn2, bn2, We1, be1, We2, be2, Wc, bc,
           Wr1, br1, Wr2, br2):
    src = edge_index[0].astype(jnp.int32)
    dst = edge_index[1].astype(jnp.int32)
    etype = edge_type.astype(jnp.int32)
    nt2 = node_type.astype(jnp.int32).reshape(N_NODES, 1)
    dist3 = dist.reshape(N_EDGES // _BE, 1, _BE)
    zeros_nd = jnp.zeros((N_NODES, DIM), F32)

    t_all = _tprep(edge_emb, We1, be1)
    t_lay = [t_all[:, i * DIM:(i + 1) * DIM] for i in range(N_CONV)]

    h, hn = _hn_first(nt2, node_emb, Wn1[0], bn1[0], Wn2[0], bn2[0])
    te = _sc_gather(t_lay[0], etype, window=128)
    for i in range(N_CONV):
        hs = _sc_gather(hn, src, window=128)
        te_next = (_sc_gather(t_lay[i + 1], etype, window=128)
                   if i + 1 < N_CONV else None)
        w1b = We1[i, DIM:DIM + N_CENTERS, :]
        m = _messages(i, te, dist3, hs, w1b, We2[i], be2[i], Wc[i], bc[i])
        te = te_next
        partials = _sc_segment_partials(m, dst, zeros_nd)
        if i + 1 < N_CONV:
            h, hn = _hn_next(h, partials, Wn1[i + 1], bn1[i + 1],
                             Wn2[i + 1], bn2[i + 1])
    out = _readout(h, partials, Wr1, br1, Wr2, br2)
    return out.reshape(-1)


# speedup vs baseline: 1.3539x; 1.1628x over previous
import functools

import jax
import jax.numpy as jnp
from jax import lax
from jax.experimental import pallas as pl
from jax.experimental.pallas import tpu as pltpu
from jax.experimental.pallas import tpu_sc as plsc

N_NODES = 10000
N_EDGES = 320000
DIM = 128
N_TYPES = 100
E_TYPES = 400
N_CENTERS = 30
CUT_LOW, CUT_HIGH = 0.0, 10.0
N_CONV = 3

F32 = jnp.float32

_MESH = plsc.VectorSubcoreMesh(core_axis_name="core", subcore_axis_name="subcore")
NC = 2
NS = 16
NW = NC * NS


def _sc_gather(table, idx_1d, window):
    b_total = idx_1d.shape[0]
    d = table.shape[1]
    idx2 = idx_1d.reshape(1, b_total)

    @functools.partial(
        pl.kernel,
        out_type=jax.ShapeDtypeStruct((b_total, d), table.dtype),
        mesh=_MESH,
    )
    def k(tab_hbm, i_hbm, o_hbm):
        def body(i_vmem, o_vmem):
            pltpu.sync_copy(tab_hbm.at[i_vmem.at[0]], o_vmem)

        pltpu.emit_pipeline(
            body,
            grid=(b_total // window,),
            in_specs=[pl.BlockSpec((1, window), lambda i: (0, i))],
            out_specs=[pl.BlockSpec((window, d), lambda i: (i, 0))],
            core_axis_name=("core", "subcore"),
            dimension_semantics=(pltpu.PARALLEL,),
        )(i_hbm, o_hbm)

    return k(table, idx2)


def _sc_segment_partials(m, dst, zeros_nd):
    ch = 80
    per_w = N_EDGES // NW
    nch = per_w // ch
    rps_main, rps_last = 640, N_NODES - 15 * 640

    @functools.partial(
        pl.kernel,
        out_type=jax.ShapeDtypeStruct((NC, N_NODES, DIM), F32),
        mesh=_MESH,
        scratch_types=[
            pltpu.VMEM((ch, DIM), F32),
            pltpu.VMEM((ch,), jnp.int32),
            pltpu.VMEM((ch, DIM), F32),
            pltpu.VMEM((ch,), jnp.int32),
            pltpu.VMEM_SHARED((N_NODES, DIM), F32),
            pltpu.SemaphoreType.DMA,
            pltpu.SemaphoreType.DMA,
            pltpu.SemaphoreType.DMA,
            pltpu.SemaphoreType.DMA,
        ],
    )
    def k(m_hbm, dst_hbm, z_hbm, out_hbm, rows_v, idx_v, rows_w, idx_w,
          acc_sh, sem_r0, sem_i0, sem_r1, sem_i1):
        cid = lax.axis_index("core")
        sid = lax.axis_index("subcore")
        wid = cid * NS + sid
        @pl.when(sid < 15)
        def _():
            pltpu.sync_copy(
                z_hbm.at[pl.ds(sid * rps_main, rps_main)],
                acc_sh.at[pl.ds(sid * rps_main, rps_main)],
            )

        @pl.when(sid == 15)
        def _():
            pltpu.sync_copy(
                z_hbm.at[pl.ds(15 * rps_main, rps_last)],
                acc_sh.at[pl.ds(15 * rps_main, rps_last)],
            )

        plsc.subcore_barrier()

        base = wid * per_w

        def _load(j, rv, iv, sr, si):
            off = base + j * ch
            pltpu.async_copy(m_hbm.at[pl.ds(off, ch)], rv, sr)
            pltpu.async_copy(dst_hbm.at[pl.ds(off, ch)], iv, si)

        def _wait(rv, iv, sr, si):
            pltpu.make_async_copy(m_hbm.at[pl.ds(base, ch)], rv, sr).wait()
            pltpu.make_async_copy(dst_hbm.at[pl.ds(base, ch)], iv, si).wait()

        _load(0, rows_v, idx_v, sem_r0, sem_i0)
        _load(1, rows_w, idx_w, sem_r1, sem_i1)

        npair = (nch - 1) // 2

        @pl.loop(0, npair)
        def _(jj):
            _wait(rows_v, idx_v, sem_r0, sem_i0)
            pltpu.sync_copy(rows_v, acc_sh.at[idx_v], add=True)
            _load(2 * jj + 2, rows_v, idx_v, sem_r0, sem_i0)
            _wait(rows_w, idx_w, sem_r1, sem_i1)
            pltpu.sync_copy(rows_w, acc_sh.at[idx_w], add=True)

            @pl.when(2 * jj + 3 < nch)
            def _():
                _load(2 * jj + 3, rows_w, idx_w, sem_r1, sem_i1)

        _wait(rows_v, idx_v, sem_r0, sem_i0)
        pltpu.sync_copy(rows_v, acc_sh.at[idx_v], add=True)

        plsc.subcore_barrier()

        @pl.when(sid < 15)
        def _():
            pltpu.sync_copy(
                acc_sh.at[pl.ds(sid * rps_main, rps_main)],
                out_hbm.at[cid].at[pl.ds(sid * rps_main, rps_main)],
            )

        @pl.when(sid == 15)
        def _():
            pltpu.sync_copy(
                acc_sh.at[pl.ds(15 * rps_main, rps_last)],
                out_hbm.at[cid].at[pl.ds(15 * rps_main, rps_last)],
            )

    return k(m, dst, zeros_nd)


def _tprep(edge_emb, We1, be1):
    def body(ee_ref, we1_ref, be1_ref, out_ref):
        ee = ee_ref[...]
        for i in range(N_CONV):
            t = jnp.dot(ee, we1_ref[i, :DIM, :], preferred_element_type=F32)
            out_ref[:, i * DIM:(i + 1) * DIM] = t + be1_ref[i, :][None, :]

    return pl.pallas_call(
        body,
        out_shape=jax.ShapeDtypeStruct((E_TYPES, N_CONV * DIM), F32),
    )(edge_emb, We1, be1)


def _hn_first(node_type_2d, node_emb, Wn1, bn1, Wn2, bn2):
    def body(nt_ref, emb_ref, w1_ref, b1_ref, w2_ref, b2_ref, h_ref, hn_ref):
        nt = nt_ref[...]
        ids = lax.broadcasted_iota(jnp.int32, (1, N_TYPES), 1)
        oh = (nt == ids).astype(F32)
        h0 = jnp.dot(oh, emb_ref[...], preferred_element_type=F32,
                     precision=jax.lax.Precision.HIGHEST)
        t = jnp.maximum(jnp.dot(h0, w1_ref[...], preferred_element_type=F32)
                        + b1_ref[...], 0.0)
        hn = jnp.dot(t, w2_ref[...], preferred_element_type=F32) + b2_ref[...]
        h_ref[...] = h0
        hn_ref[...] = hn

    return pl.pallas_call(
        body,
        out_shape=(
            jax.ShapeDtypeStruct((N_NODES, DIM), F32),
            jax.ShapeDtypeStruct((N_NODES, DIM), F32),
        ),
    )(node_type_2d, node_emb, Wn1, bn1.reshape(1, DIM), Wn2, bn2.reshape(1, DIM))


def _hn_next(h_prev, partials, Wn1, bn1, Wn2, bn2):
    def body(h_ref, p_ref, w1_ref, b1_ref, w2_ref, b2_ref, h_out, hn_ref):
        h = h_ref[...] + p_ref[0] + p_ref[1]
        t = jnp.maximum(jnp.dot(h, w1_ref[...], preferred_element_type=F32)
                        + b1_ref[...], 0.0)
        hn = jnp.dot(t, w2_ref[...], preferred_element_type=F32) + b2_ref[...]
        h_out[...] = h
        hn_ref[...] = hn

    return pl.pallas_call(
        body,
        out_shape=(
            jax.ShapeDtypeStruct((N_NODES, DIM), F32),
            jax.ShapeDtypeStruct((N_NODES, DIM), F32),
        ),
    )(h_prev, partials, Wn1, bn1.reshape(1, DIM), Wn2, bn2.reshape(1, DIM))


_BE = 1280


def _messages(layer, te_all, dist3, hs, w1b, We2, be2, Wc, bc):
    gap = (CUT_HIGH - CUT_LOW) / N_CENTERS
    inv_gap2 = 1.0 / (gap * gap)
    BF = jnp.bfloat16

    def body(te_ref, d_ref, hs_ref, w1b_ref, w2_ref, b2_ref, wc_ref, bc_ref,
             m_ref):
        step = (CUT_HIGH - CUT_LOW) / (N_CENTERS - 1)
        centers_t = (lax.broadcasted_iota(jnp.int32, (N_CENTERS, 1), 0)
                     .astype(F32) * step + CUT_LOW)
        d = d_ref[0]
        delta = d - centers_t
        rbf_t = jnp.exp(-(delta * delta) * inv_gap2)
        r = lax.dot_general(rbf_t, w1b_ref[...],
                            (((0,), (0,)), ((), ())),
                            preferred_element_type=F32)
        pre = te_ref[...] + r
        he = jnp.dot(jnp.maximum(pre, 0.0), w2_ref[...],
                     preferred_element_type=F32) + b2_ref[...]
        prod = hs_ref[...] * he
        m_ref[...] = jnp.tanh(
            jnp.dot(prod, wc_ref[...],
                    preferred_element_type=F32) + bc_ref[...])

    nb = N_EDGES // _BE
    full = lambda *s: pl.BlockSpec(s, lambda b: tuple(0 for _ in s))
    return pl.pallas_call(
        body,
        grid=(nb,),
        in_specs=[
            pl.BlockSpec((_BE, DIM), lambda b: (b, 0)),
            pl.BlockSpec((1, 1, _BE), lambda b: (b, 0, 0)),
            pl.BlockSpec((_BE, DIM), lambda b: (b, 0)),
            full(N_CENTERS, DIM),
            full(DIM, DIM),
            full(1, DIM),
            full(DIM, DIM),
            full(1, DIM),
        ],
        out_specs=pl.BlockSpec((_BE, DIM), lambda b: (b, 0)),
        out_shape=jax.ShapeDtypeStruct((N_EDGES, DIM), F32),
    )(te_all, dist3, hs, w1b, We2, be2.reshape(1, DIM), Wc, bc.reshape(1, DIM))


def _readout(h_prev, partials, Wr1, br1, Wr2, br2):
    def body(h_ref, p_ref, w1_ref, b1_ref, w2_ref, b2_ref, out_ref):
        h = h_ref[...] + p_ref[0] + p_ref[1]
        t = jnp.maximum(jnp.dot(h, w1_ref[...], preferred_element_type=F32)
                        + b1_ref[...], 0.0)
        r = jnp.dot(t, w2_ref[...], preferred_element_type=F32) + b2_ref[...]
        out_ref[...] = jnp.sum(r, axis=0, keepdims=True)

    return pl.pallas_call(
        body,
        out_shape=jax.ShapeDtypeStruct((1, 1), F32),
    )(h_prev, partials, Wr1, br1.reshape(1, DIM), Wr2, br2.reshape(1, 1))


def kernel(node_type, edge_type, edge_index, dist, node_emb, edge_emb,
           Wn1, bn1, Wn2, bn2, We1, be1, We2, be2, Wc, bc,
           Wr1, br1, Wr2, br2):
    src = edge_index[0].astype(jnp.int32)
    dst = edge_index[1].astype(jnp.int32)
    etype = edge_type.astype(jnp.int32)
    nt2 = node_type.astype(jnp.int32).reshape(N_NODES, 1)
    dist3 = dist.reshape(N_EDGES // _BE, 1, _BE)
    zeros_nd = jnp.zeros((N_NODES, DIM), F32)

    t_all = _tprep(edge_emb, We1, be1)
    t_lay = [t_all[:, i * DIM:(i + 1) * DIM] for i in range(N_CONV)]

    h, hn = _hn_first(nt2, node_emb, Wn1[0], bn1[0], Wn2[0], bn2[0])
    te = _sc_gather(t_lay[0], etype, window=128)
    for i in range(N_CONV):
        hs = _sc_gather(hn, src, window=128)
        te_next = (_sc_gather(t_lay[i + 1], etype, window=128)
                   if i + 1 < N_CONV else None)
        w1b = We1[i, DIM:DIM + N_CENTERS, :]
        m = _messages(i, te, dist3, hs, w1b, We2[i], be2[i], Wc[i], bc[i])
        te = te_next
        partials = _sc_segment_partials(m, dst, zeros_nd)
        if i + 1 < N_CONV:
            h, hn = _hn_next(h, partials, Wn1[i + 1], bn1[i + 1],
                             Wn2[i + 1], bn2[i + 1])
    out = _readout(h, partials, Wr1, br1, Wr2, br2)
    return out.reshape(-1)
